# baseline (speedup 1.0000x reference)
import functools

import jax
import jax.numpy as jnp
from jax import lax
from jax.experimental import pallas as pl
from jax.experimental.pallas import tpu as pltpu
from jax.experimental.pallas import tpu_sc as plsc

_E = 8
_K = 2
_BLK = 512
_NW = 32
_NBLK_PAD = 80


def _router_body(x_ref, gw_ref, ti_ref, tw_ref, xbi_ref, hexcl_ref, tot_ref,
                 acc_ref):
    t = pl.program_id(0)
    x = x_ref[...]
    logits = jnp.dot(x, gw_ref[...], preferred_element_type=jnp.float32)
    p = jax.nn.softmax(logits, axis=-1)
    n_e = p.shape[-1]
    iota = lax.broadcasted_iota(jnp.int32, p.shape, dimension=1)
    m1 = jnp.max(p, axis=-1, keepdims=True)
    idx1 = jnp.min(jnp.where(p == m1, iota, n_e), axis=-1, keepdims=True)
    first1 = iota == idx1
    p2 = jnp.where(first1, -jnp.inf, p)
    m2 = jnp.max(p2, axis=-1, keepdims=True)
    idx2 = jnp.min(jnp.where(p2 == m2, iota, n_e), axis=-1, keepdims=True)
    first2 = iota == idx2
    sel = first1 | first2

    denom = m1 + m2
    tw_ref[...] = jnp.concatenate([m1 / denom, m2 / denom], axis=1)
    ti_ref[...] = jnp.concatenate([idx1, idx2], axis=1)

    xb = x.astype(jnp.bfloat16)
    half = xb.shape[1] // 2
    ah = lax.bitcast_convert_type(xb[:, :half], jnp.uint16).astype(jnp.uint32)
    al = lax.bitcast_convert_type(xb[:, half:], jnp.uint16).astype(jnp.uint32)
    xbi_ref[...] = lax.bitcast_convert_type((ah << 16) | al, jnp.int32)

    counts = jnp.sum(sel.astype(jnp.int32), axis=0, keepdims=True)
    counts16 = jnp.pad(counts, ((0, 0), (0, 8)))

    @pl.when(t == 0)
    def _():
        acc_ref[...] = jnp.zeros_like(acc_ref)

    hexcl_ref[...] = acc_ref[...].reshape(1, 1, 16)
    acc_ref[...] += counts16
    tot_ref[...] = acc_ref[...].reshape(1, 1, 16)


def _run_router(x, gw, T, H):
    n_t = T // _BLK
    return pl.pallas_call(
        _router_body,
        grid=(n_t,),
        in_specs=[
            pl.BlockSpec((_BLK, H), lambda t: (t, 0)),
            pl.BlockSpec((H, _E), lambda t: (0, 0)),
        ],
        out_specs=[
            pl.BlockSpec((_BLK, _K), lambda t: (t, 0)),
            pl.BlockSpec((_BLK, _K), lambda t: (t, 0)),
            pl.BlockSpec((_BLK, H // 2), lambda t: (t, 0)),
            pl.BlockSpec((1, 1, 16), lambda t: (t, 0, 0)),
            pl.BlockSpec((1, 1, 16), lambda t: (0, 0, 0)),
        ],
        out_shape=[
            jax.ShapeDtypeStruct((T, _K), jnp.int32),
            jax.ShapeDtypeStruct((T, _K), jnp.float32),
            jax.ShapeDtypeStruct((T, H // 2), jnp.int32),
            jax.ShapeDtypeStruct((n_t, 1, 16), jnp.int32),
            jax.ShapeDtypeStruct((1, 1, 16), jnp.int32),
        ],
        scratch_shapes=[pltpu.VMEM((1, 16), jnp.int32)],
    )(x, gw)


def _vgather(v, idx):
    dn = lax.GatherDimensionNumbers(
        offset_dims=(), collapsed_slice_dims=(0,), start_index_map=(0,))
    return lax.gather(v, idx[:, None], dn, slice_sizes=(1,),
                      mode=lax.GatherScatterMode.PROMISE_IN_BOUNDS)


def _prefix_sum_inc(x, lane):
    for sh in (1, 2, 4, 8):
        shifted = _vgather(x, jnp.maximum(lane - sh, 0))
        x = x + jnp.where(lane >= sh, shifted, 0)
    return x


def _make_dispatch(T, H, P):
    tpw = T // _NW
    ppw = tpw * _K
    mesh = plsc.VectorSubcoreMesh(core_axis_name="c", subcore_axis_name="s")

    @functools.partial(
        pl.kernel,
        mesh=mesh,
        out_type=[
            jax.ShapeDtypeStruct((P, H // 2), jnp.int32),
            jax.ShapeDtypeStruct((T,), jnp.int32),
            jax.ShapeDtypeStruct((T,), jnp.int32),
            jax.ShapeDtypeStruct((_NBLK_PAD,), jnp.int32),
        ],
        scratch_types=[
            pltpu.VMEM((T * _K // _NW,), jnp.int32),
            pltpu.VMEM((T // _NW,), jnp.int32),
            pltpu.VMEM((T // _NW,), jnp.int32),
            pltpu.VMEM((T // _NW,), jnp.int32),
            pltpu.VMEM((T // _NW,), jnp.int32),
            pltpu.VMEM((16 * 33,), jnp.int32),
            pltpu.VMEM((16, H // 2), jnp.int32),
            pltpu.VMEM((_NBLK_PAD,), jnp.int32),
            pltpu.SemaphoreType.DMA,
            pltpu.SemaphoreType.DMA,
        ],
    )
    def dispatch(eid_hbm, hexcl_hbm, tot_hbm, x_hbm,
                 xs_hbm, pos0_hbm, pos1_hbm, bexp_hbm,
                 eid_v, e0_v, e1_v, p0_v, p1_v, hist_v, rows_v, bexp_v, sem,
                 sem2):
        w = lax.axis_index("s") * 2 + lax.axis_index("c")
        lane = lax.iota(jnp.int32, 16)
        lm8 = lane < 8

        pltpu.sync_copy(eid_hbm.at[pl.ds(w * ppw, ppw)], eid_v)
        pltpu.sync_copy(hexcl_hbm, hist_v.at[pl.ds(0, 512)])
        pltpu.sync_copy(tot_hbm, hist_v.at[pl.ds(512, 16)])

        ge = jnp.minimum(2 * lane, 15)
        go = jnp.minimum(2 * lane + 1, 15)
        gsh = jnp.maximum(2 * lane - 16, 0)
        gsh1 = jnp.maximum(2 * lane - 15, 0)

        def deint_body(j, _):
            va = eid_v[pl.ds(32 * j, 16)]
            vb = eid_v[pl.ds(32 * j + 16, 16)]
            e0_v[pl.ds(16 * j, 16)] = jnp.where(
                lm8, _vgather(va, ge), _vgather(vb, gsh))
            e1_v[pl.ds(16 * j, 16)] = jnp.where(
                lm8, _vgather(va, go), _vgather(vb, gsh1))
            return 0

        lax.fori_loop(0, tpw // 16, deint_body, 0)

        base_v = jnp.where(lm8, hist_v[pl.ds(w * 16, 16)], 0)
        tot_v = jnp.where(lm8, hist_v[pl.ds(512, 16)], 0)
        cnt_p = lax.shift_left(
            lax.shift_right_logical(tot_v + (_BLK - 1), 9), 9)
        inc = _prefix_sum_inc(cnt_p, lane)
        off_excl = inc - cnt_p
        cnt_vec0 = off_excl + base_v

        def make_pos_body(src_ref, dst_ref):
            def pos_body(v, cnt_vec):
                ev = src_ref[pl.ds(v * 16, 16)]
                pos = jnp.zeros(16, jnp.int32)
                for e in range(_E):
                    m = ev == e
                    r = _prefix_sum_inc(jnp.where(m, 1, 0), lane)
                    base_e = _vgather(cnt_vec, jnp.full(16, e, jnp.int32))
                    pos = jnp.where(m, base_e + r - 1, pos)
                    r_tot = _vgather(r, jnp.full(16, 15, jnp.int32))
                    cnt_vec = jnp.where(lane == e, cnt_vec + r_tot, cnt_vec)
                dst_ref[pl.ds(v * 16, 16)] = pos
                return cnt_vec
            return pos_body

        cnt_vec1 = lax.fori_loop(0, tpw // 16, make_pos_body(e0_v, p0_v),
                                 cnt_vec0)
        lax.fori_loop(0, tpw // 16, make_pos_body(e1_v, p1_v), cnt_vec1)

        pltpu.sync_copy(p0_v, pos0_hbm.at[pl.ds(w * tpw, tpw)])
        pltpu.sync_copy(p1_v, pos1_hbm.at[pl.ds(w * tpw, tpw)])

        def scat_body(m, _):
            pltpu.sync_copy(x_hbm.at[pl.ds(w * tpw + m * 16, 16)], rows_v)
            i0 = p0_v[pl.ds(m * 16, 16)]
            i1 = p1_v[pl.ds(m * 16, 16)]
            c0 = pltpu.async_copy(rows_v, xs_hbm.at[i0], sem)
            c1 = pltpu.async_copy(rows_v, xs_hbm.at[i1], sem2)
            c0.wait()
            c1.wait()
            return 0

        lax.fori_loop(0, tpw // 16, scat_body, 0)

        @pl.when(w == 0)
        def _():
            for mb in range(_NBLK_PAD // 16):
                bstart = (lane + 16 * mb) * _BLK
                acc = jnp.zeros(16, jnp.int32)
                for e in range(_E):
                    pend_e = _vgather(inc, jnp.full(16, e, jnp.int32))
                    acc += jnp.where(bstart >= pend_e, 1, 0)
                bexp_v[pl.ds(mb * 16, 16)] = jnp.minimum(acc, _E - 1)
            pltpu.sync_copy(bexp_v, bexp_hbm)

    return dispatch


def _gmm_body(bexp_ref, xs_ref, gu_ref, dn_ref, os_ref):
    vu = lax.bitcast_convert_type(xs_ref[...], jnp.uint32)
    ah = lax.bitcast_convert_type((vu >> 16).astype(jnp.uint16), jnp.bfloat16)
    al = lax.bitcast_convert_type(vu.astype(jnp.uint16), jnp.bfloat16)
    half_k = ah.shape[1]
    gu = (jnp.dot(ah, gu_ref[0, :half_k], preferred_element_type=jnp.float32)
          + jnp.dot(al, gu_ref[0, half_k:],
                    preferred_element_type=jnp.float32))
    inter = gu.shape[-1] // 2
    gate = gu[:, :inter]
    up = gu[:, inter:]
    h = (gate * lax.logistic(gate)) * up
    o = jnp.dot(h.astype(jnp.bfloat16), dn_ref[0],
                preferred_element_type=jnp.float32)
    ob = o.astype(jnp.bfloat16)
    half = ob.shape[1] // 2
    oh = lax.bitcast_convert_type(ob[:, :half], jnp.uint16).astype(jnp.uint32)
    ol = lax.bitcast_convert_type(ob[:, half:], jnp.uint16).astype(jnp.uint32)
    os_ref[...] = lax.bitcast_convert_type((oh << 16) | ol, jnp.int32)


def _run_gmm(xs, gu_b, dn_b, bexp, P, H, I2):
    n_blk = P // _BLK
    grid_spec = pltpu.PrefetchScalarGridSpec(
        num_scalar_prefetch=1,
        grid=(n_blk,),
        in_specs=[
            pl.BlockSpec((_BLK, H // 2), lambda i, s: (i, 0)),
            pl.BlockSpec((1, H, I2), lambda i, s: (s[i], 0, 0)),
            pl.BlockSpec((1, I2 // 2, H), lambda i, s: (s[i], 0, 0)),
        ],
        out_specs=pl.BlockSpec((_BLK, H // 2), lambda i, s: (i, 0)),
    )
    return pl.pallas_call(
        _gmm_body,
        grid_spec=grid_spec,
        out_shape=jax.ShapeDtypeStruct((P, H // 2), jnp.int32),
    )(bexp, xs, gu_b, dn_b)


def _make_gather(T, H, P):
    tpw = T // _NW
    mesh = plsc.VectorSubcoreMesh(core_axis_name="c", subcore_axis_name="s")

    @functools.partial(
        pl.kernel,
        mesh=mesh,
        out_type=jax.ShapeDtypeStruct((_K, T, H // 2), jnp.int32),
        scratch_types=[
            pltpu.VMEM((T // _NW,), jnp.int32),
            pltpu.VMEM((T // _NW,), jnp.int32),
            pltpu.VMEM((16, H // 2), jnp.int32),
            pltpu.VMEM((16, H // 2), jnp.int32),
            pltpu.SemaphoreType.DMA,
            pltpu.SemaphoreType.DMA,
        ],
    )
    def gather(os_hbm, pos0_hbm, pos1_hbm, op_hbm, p0_v, p1_v, r0_v, r1_v,
               sem0, sem1):
        w = lax.axis_index("s") * 2 + lax.axis_index("c")
        pltpu.sync_copy(pos0_hbm.at[pl.ds(w * tpw, tpw)], p0_v)
        pltpu.sync_copy(pos1_hbm.at[pl.ds(w * tpw, tpw)], p1_v)

        def body(m, _):
            i0 = p0_v[pl.ds(m * 16, 16)]
            i1 = p1_v[pl.ds(m * 16, 16)]
            c0 = pltpu.async_copy(os_hbm.at[i0], r0_v, sem0)
            c1 = pltpu.async_copy(os_hbm.at[i1], r1_v, sem1)
            c0.wait()
            pltpu.sync_copy(r0_v, op_hbm.at[0, pl.ds(w * tpw + m * 16, 16)])
            c1.wait()
            pltpu.sync_copy(r1_v, op_hbm.at[1, pl.ds(w * tpw + m * 16, 16)])
            return 0

        lax.fori_loop(0, tpw // 16, body, 0)

    return gather


def _combine_body(op0_ref, op1_ref, tw_ref, out_ref):
    hmask = jnp.int32(-65536)
    v0 = op0_ref[0]
    v1 = op1_ref[0]
    f0h = lax.bitcast_convert_type(v0 & hmask, jnp.float32)
    f1h = lax.bitcast_convert_type(v1 & hmask, jnp.float32)
    f0l = lax.bitcast_convert_type(lax.shift_left(v0, 16), jnp.float32)
    f1l = lax.bitcast_convert_type(lax.shift_left(v1, 16), jnp.float32)
    tw = tw_ref[...]
    w0 = tw[:, 0:1]
    w1 = tw[:, 1:2]
    out_ref[...] = jnp.concatenate(
        [f0h * w0 + f1h * w1, f0l * w0 + f1l * w1], axis=1)


def _run_combine(op, tw, T, H):
    return pl.pallas_call(
        _combine_body,
        grid=(T // _BLK,),
        in_specs=[
            pl.BlockSpec((1, _BLK, H // 2), lambda t: (0, t, 0)),
            pl.BlockSpec((1, _BLK, H // 2), lambda t: (1, t, 0)),
            pl.BlockSpec((_BLK, _K), lambda t: (t, 0)),
        ],
        out_specs=pl.BlockSpec((_BLK, H), lambda t: (t, 0)),
        out_shape=jax.ShapeDtypeStruct((T, H), jnp.float32),
    )(op, op, tw)


@jax.jit
def kernel(hidden_states, gate_up_proj, down_proj, gate_weight):
    B, S, H = hidden_states.shape
    E, _, I2 = gate_up_proj.shape
    T = B * S
    P = (T * _K // _BLK + E) * _BLK

    x = hidden_states.reshape(T, H)
    gu_b = gate_up_proj.astype(jnp.bfloat16)
    dn_b = down_proj.astype(jnp.bfloat16)

    top_i, top_w, xbi, hexcl, tot = _run_router(x, gate_weight, T, H)

    xs, pos0, pos1, bexp = _make_dispatch(T, H, P)(
        top_i.reshape(-1), hexcl.reshape(-1), tot.reshape(-1), xbi)

    os_ = _run_gmm(xs, gu_b, dn_b, bexp, P, H, I2)

    op = _make_gather(T, H, P)(os_, pos0, pos1)
    out = _run_combine(op, top_w, T, H)
    return out.reshape(B, S, H)


# speedup vs baseline: 1.0191x; 1.0191x over previous
import functools

import jax
import jax.numpy as jnp
from jax import lax
from jax.experimental import pallas as pl
from jax.experimental.pallas import tpu as pltpu
from jax.experimental.pallas import tpu_sc as plsc

_E = 8
_K = 2
_BLK = 512
_NW = 32
_NBLK_PAD = 80


def _router_body(x_ref, gw_ref, ti_ref, tw_ref, xbi_ref, hexcl_ref, tot_ref,
                 acc_ref):
    t = pl.program_id(0)
    x = x_ref[...]
    logits = jnp.dot(x, gw_ref[...], preferred_element_type=jnp.float32)
    p = jax.nn.softmax(logits, axis=-1)
    n_e = p.shape[-1]
    iota = lax.broadcasted_iota(jnp.int32, p.shape, dimension=1)
    m1 = jnp.max(p, axis=-1, keepdims=True)
    idx1 = jnp.min(jnp.where(p == m1, iota, n_e), axis=-1, keepdims=True)
    first1 = iota == idx1
    p2 = jnp.where(first1, -jnp.inf, p)
    m2 = jnp.max(p2, axis=-1, keepdims=True)
    idx2 = jnp.min(jnp.where(p2 == m2, iota, n_e), axis=-1, keepdims=True)
    first2 = iota == idx2
    sel = first1 | first2

    denom = m1 + m2
    tw_ref[...] = jnp.concatenate([m1 / denom, m2 / denom], axis=1)
    ti_ref[...] = jnp.concatenate([idx1, idx2], axis=1)

    xb = x.astype(jnp.bfloat16)
    half = xb.shape[1] // 2
    ah = lax.bitcast_convert_type(xb[:, :half], jnp.uint16).astype(jnp.uint32)
    al = lax.bitcast_convert_type(xb[:, half:], jnp.uint16).astype(jnp.uint32)
    xbi_ref[...] = lax.bitcast_convert_type((ah << 16) | al, jnp.int32)

    counts = jnp.sum(sel.astype(jnp.int32), axis=0, keepdims=True)
    counts16 = jnp.pad(counts, ((0, 0), (0, 8)))

    @pl.when(t == 0)
    def _():
        acc_ref[...] = jnp.zeros_like(acc_ref)

    hexcl_ref[...] = acc_ref[...].reshape(1, 1, 16)
    acc_ref[...] += counts16
    tot_ref[...] = acc_ref[...].reshape(1, 1, 16)


def _run_router(x, gw, T, H):
    n_t = T // _BLK
    return pl.pallas_call(
        _router_body,
        grid=(n_t,),
        in_specs=[
            pl.BlockSpec((_BLK, H), lambda t: (t, 0)),
            pl.BlockSpec((H, _E), lambda t: (0, 0)),
        ],
        out_specs=[
            pl.BlockSpec((_BLK, _K), lambda t: (t, 0)),
            pl.BlockSpec((_BLK, _K), lambda t: (t, 0)),
            pl.BlockSpec((_BLK, H // 2), lambda t: (t, 0)),
            pl.BlockSpec((1, 1, 16), lambda t: (t, 0, 0)),
            pl.BlockSpec((1, 1, 16), lambda t: (0, 0, 0)),
        ],
        out_shape=[
            jax.ShapeDtypeStruct((T, _K), jnp.int32),
            jax.ShapeDtypeStruct((T, _K), jnp.float32),
            jax.ShapeDtypeStruct((T, H // 2), jnp.int32),
            jax.ShapeDtypeStruct((n_t, 1, 16), jnp.int32),
            jax.ShapeDtypeStruct((1, 1, 16), jnp.int32),
        ],
        scratch_shapes=[pltpu.VMEM((1, 16), jnp.int32)],
    )(x, gw)


def _vgather(v, idx):
    dn = lax.GatherDimensionNumbers(
        offset_dims=(), collapsed_slice_dims=(0,), start_index_map=(0,))
    return lax.gather(v, idx[:, None], dn, slice_sizes=(1,),
                      mode=lax.GatherScatterMode.PROMISE_IN_BOUNDS)


def _prefix_sum_inc(x, lane):
    for sh in (1, 2, 4, 8):
        shifted = _vgather(x, jnp.maximum(lane - sh, 0))
        x = x + jnp.where(lane >= sh, shifted, 0)
    return x


def _make_dispatch(T, H, P):
    tpw = T // _NW
    ppw = tpw * _K
    mesh = plsc.VectorSubcoreMesh(core_axis_name="c", subcore_axis_name="s")

    @functools.partial(
        pl.kernel,
        mesh=mesh,
        out_type=[
            jax.ShapeDtypeStruct((P, H // 2), jnp.int32),
            jax.ShapeDtypeStruct((T,), jnp.int32),
            jax.ShapeDtypeStruct((T,), jnp.int32),
            jax.ShapeDtypeStruct((_NBLK_PAD,), jnp.int32),
        ],
        scratch_types=[
            pltpu.VMEM((T * _K // _NW,), jnp.int32),
            pltpu.VMEM((T // _NW,), jnp.int32),
            pltpu.VMEM((T // _NW,), jnp.int32),
            pltpu.VMEM((T // _NW,), jnp.int32),
            pltpu.VMEM((T // _NW,), jnp.int32),
            pltpu.VMEM((16 * 33,), jnp.int32),
            pltpu.VMEM((16, H // 2), jnp.int32),
            pltpu.VMEM((16, H // 2), jnp.int32),
            pltpu.VMEM((_NBLK_PAD,), jnp.int32),
            pltpu.SemaphoreType.DMA,
            pltpu.SemaphoreType.DMA,
            pltpu.SemaphoreType.DMA,
            pltpu.SemaphoreType.DMA,
        ],
    )
    def dispatch(eid_hbm, hexcl_hbm, tot_hbm, x_hbm,
                 xs_hbm, pos0_hbm, pos1_hbm, bexp_hbm,
                 eid_v, e0_v, e1_v, p0_v, p1_v, hist_v, rows_a, rows_b,
                 bexp_v, sem, sem2, semra, semrb):
        w = lax.axis_index("s") * 2 + lax.axis_index("c")
        lane = lax.iota(jnp.int32, 16)
        lm8 = lane < 8

        pltpu.sync_copy(eid_hbm.at[pl.ds(w * ppw, ppw)], eid_v)
        pltpu.sync_copy(hexcl_hbm, hist_v.at[pl.ds(0, 512)])
        pltpu.sync_copy(tot_hbm, hist_v.at[pl.ds(512, 16)])

        ge = jnp.minimum(2 * lane, 15)
        go = jnp.minimum(2 * lane + 1, 15)
        gsh = jnp.maximum(2 * lane - 16, 0)
        gsh1 = jnp.maximum(2 * lane - 15, 0)

        def deint_body(j, _):
            va = eid_v[pl.ds(32 * j, 16)]
            vb = eid_v[pl.ds(32 * j + 16, 16)]
            e0_v[pl.ds(16 * j, 16)] = jnp.where(
                lm8, _vgather(va, ge), _vgather(vb, gsh))
            e1_v[pl.ds(16 * j, 16)] = jnp.where(
                lm8, _vgather(va, go), _vgather(vb, gsh1))
            return 0

        lax.fori_loop(0, tpw // 16, deint_body, 0)

        base_v = jnp.where(lm8, hist_v[pl.ds(w * 16, 16)], 0)
        tot_v = jnp.where(lm8, hist_v[pl.ds(512, 16)], 0)
        cnt_p = lax.shift_left(
            lax.shift_right_logical(tot_v + (_BLK - 1), 9), 9)
        inc = _prefix_sum_inc(cnt_p, lane)
        off_excl = inc - cnt_p
        cnt_vec0 = off_excl + base_v

        def make_pos_body(src_ref, dst_ref):
            def pos_body(v, cnt_vec):
                ev = src_ref[pl.ds(v * 16, 16)]
                pos = jnp.zeros(16, jnp.int32)
                for e in range(_E):
                    m = ev == e
                    r = _prefix_sum_inc(jnp.where(m, 1, 0), lane)
                    base_e = _vgather(cnt_vec, jnp.full(16, e, jnp.int32))
                    pos = jnp.where(m, base_e + r - 1, pos)
                    r_tot = _vgather(r, jnp.full(16, 15, jnp.int32))
                    cnt_vec = jnp.where(lane == e, cnt_vec + r_tot, cnt_vec)
                dst_ref[pl.ds(v * 16, 16)] = pos
                return cnt_vec
            return pos_body

        cnt_vec1 = lax.fori_loop(0, tpw // 16, make_pos_body(e0_v, p0_v),
                                 cnt_vec0)
        lax.fori_loop(0, tpw // 16, make_pos_body(e1_v, p1_v), cnt_vec1)

        pltpu.sync_copy(p0_v, pos0_hbm.at[pl.ds(w * tpw, tpw)])
        pltpu.sync_copy(p1_v, pos1_hbm.at[pl.ds(w * tpw, tpw)])

        n_chunk = tpw // 16
        pltpu.async_copy(x_hbm.at[pl.ds(w * tpw, 16)], rows_a, semra).wait()

        def scat_body(m, _):
            for b in range(2):
                mm = 2 * m + b
                rows, semr = (rows_a, semra) if b == 0 else (rows_b, semrb)
                nrows, nsemr = (rows_b, semrb) if b == 0 else (rows_a, semra)

                @pl.when(mm + 1 < n_chunk)
                def _():
                    pltpu.async_copy(
                        x_hbm.at[pl.ds(w * tpw + (mm + 1) * 16, 16)],
                        nrows, nsemr)

                @pl.when(mm > 0)
                def _():
                    pltpu.make_async_copy(
                        x_hbm.at[pl.ds(w * tpw + mm * 16, 16)],
                        rows, semr).wait()

                i0 = p0_v[pl.ds(mm * 16, 16)]
                i1 = p1_v[pl.ds(mm * 16, 16)]
                c0 = pltpu.async_copy(rows, xs_hbm.at[i0], sem)
                c1 = pltpu.async_copy(rows, xs_hbm.at[i1], sem2)
                c0.wait()
                c1.wait()
            return 0

        lax.fori_loop(0, n_chunk // 2, scat_body, 0)

        @pl.when(w == 0)
        def _():
            for mb in range(_NBLK_PAD // 16):
                bstart = (lane + 16 * mb) * _BLK
                acc = jnp.zeros(16, jnp.int32)
                for e in range(_E):
                    pend_e = _vgather(inc, jnp.full(16, e, jnp.int32))
                    acc += jnp.where(bstart >= pend_e, 1, 0)
                bexp_v[pl.ds(mb * 16, 16)] = jnp.minimum(acc, _E - 1)
            pltpu.sync_copy(bexp_v, bexp_hbm)

    return dispatch


def _gmm_body(bexp_ref, xs_ref, gu_ref, dn_ref, os_ref):
    vu = lax.bitcast_convert_type(xs_ref[...], jnp.uint32)
    ah = lax.bitcast_convert_type((vu >> 16).astype(jnp.uint16), jnp.bfloat16)
    al = lax.bitcast_convert_type(vu.astype(jnp.uint16), jnp.bfloat16)
    xb = jnp.concatenate([ah, al], axis=1)
    gu = jnp.dot(xb, gu_ref[0], preferred_element_type=jnp.float32)
    inter = gu.shape[-1] // 2
    gate = gu[:, :inter]
    up = gu[:, inter:]
    h = (gate * lax.logistic(gate)) * up
    o = jnp.dot(h.astype(jnp.bfloat16), dn_ref[0],
                preferred_element_type=jnp.float32)
    ob = o.astype(jnp.bfloat16)
    half = ob.shape[1] // 2
    oh = lax.bitcast_convert_type(ob[:, :half], jnp.uint16).astype(jnp.uint32)
    ol = lax.bitcast_convert_type(ob[:, half:], jnp.uint16).astype(jnp.uint32)
    os_ref[...] = lax.bitcast_convert_type((oh << 16) | ol, jnp.int32)


def _run_gmm(xs, gu_b, dn_b, bexp, P, H, I2):
    n_blk = P // _BLK
    grid_spec = pltpu.PrefetchScalarGridSpec(
        num_scalar_prefetch=1,
        grid=(n_blk,),
        in_specs=[
            pl.BlockSpec((_BLK, H // 2), lambda i, s: (i, 0)),
            pl.BlockSpec((1, H, I2), lambda i, s: (s[i], 0, 0)),
            pl.BlockSpec((1, I2 // 2, H), lambda i, s: (s[i], 0, 0)),
        ],
        out_specs=pl.BlockSpec((_BLK, H // 2), lambda i, s: (i, 0)),
    )
    return pl.pallas_call(
        _gmm_body,
        grid_spec=grid_spec,
        out_shape=jax.ShapeDtypeStruct((P, H // 2), jnp.int32),
    )(bexp, xs, gu_b, dn_b)


def _make_gather(T, H, P):
    tpw = T // _NW
    mesh = plsc.VectorSubcoreMesh(core_axis_name="c", subcore_axis_name="s")

    @functools.partial(
        pl.kernel,
        mesh=mesh,
        out_type=jax.ShapeDtypeStruct((_K, T, H // 2), jnp.int32),
        scratch_types=[
            pltpu.VMEM((T // _NW,), jnp.int32),
            pltpu.VMEM((T // _NW,), jnp.int32),
            pltpu.VMEM((16, H // 2), jnp.int32),
            pltpu.VMEM((16, H // 2), jnp.int32),
            pltpu.VMEM((16, H // 2), jnp.int32),
            pltpu.VMEM((16, H // 2), jnp.int32),
            pltpu.SemaphoreType.DMA,
            pltpu.SemaphoreType.DMA,
            pltpu.SemaphoreType.DMA,
            pltpu.SemaphoreType.DMA,
        ],
    )
    def gather(os_hbm, pos0_hbm, pos1_hbm, op_hbm, p0_v, p1_v, r0a, r1a,
               r0b, r1b, s0a, s1a, s0b, s1b):
        w = lax.axis_index("s") * 2 + lax.axis_index("c")
        pltpu.sync_copy(pos0_hbm.at[pl.ds(w * tpw, tpw)], p0_v)
        pltpu.sync_copy(pos1_hbm.at[pl.ds(w * tpw, tpw)], p1_v)
        n_chunk = tpw // 16

        def fire(mm, r0, r1, s0, s1):
            i0 = p0_v[pl.ds(mm * 16, 16)]
            i1 = p1_v[pl.ds(mm * 16, 16)]
            pltpu.async_copy(os_hbm.at[i0], r0, s0)
            pltpu.async_copy(os_hbm.at[i1], r1, s1)

        def drain_and_write(mm, r0, r1, s0, s1):
            i0 = p0_v[pl.ds(mm * 16, 16)]
            i1 = p1_v[pl.ds(mm * 16, 16)]
            pltpu.make_async_copy(os_hbm.at[i0], r0, s0).wait()
            pltpu.sync_copy(r0, op_hbm.at[0, pl.ds(w * tpw + mm * 16, 16)])
            pltpu.make_async_copy(os_hbm.at[i1], r1, s1).wait()
            pltpu.sync_copy(r1, op_hbm.at[1, pl.ds(w * tpw + mm * 16, 16)])

        fire(0, r0a, r1a, s0a, s1a)

        def body(m, _):
            for b in range(2):
                mm = 2 * m + b
                cur = (r0a, r1a, s0a, s1a) if b == 0 else (r0b, r1b, s0b, s1b)
                nxt = (r0b, r1b, s0b, s1b) if b == 0 else (r0a, r1a, s0a, s1a)

                @pl.when(mm + 1 < n_chunk)
                def _():
                    fire(mm + 1, *nxt)

                drain_and_write(mm, *cur)
            return 0

        lax.fori_loop(0, n_chunk // 2, body, 0)

    return gather


def _combine_body(op0_ref, op1_ref, tw_ref, out_ref):
    hmask = jnp.int32(-65536)
    v0 = op0_ref[0]
    v1 = op1_ref[0]
    f0h = lax.bitcast_convert_type(v0 & hmask, jnp.float32)
    f1h = lax.bitcast_convert_type(v1 & hmask, jnp.float32)
    f0l = lax.bitcast_convert_type(lax.shift_left(v0, 16), jnp.float32)
    f1l = lax.bitcast_convert_type(lax.shift_left(v1, 16), jnp.float32)
    tw = tw_ref[...]
    w0 = tw[:, 0:1]
    w1 = tw[:, 1:2]
    out_ref[...] = jnp.concatenate(
        [f0h * w0 + f1h * w1, f0l * w0 + f1l * w1], axis=1)


def _run_combine(op, tw, T, H):
    return pl.pallas_call(
        _combine_body,
        grid=(T // _BLK,),
        in_specs=[
            pl.BlockSpec((1, _BLK, H // 2), lambda t: (0, t, 0)),
            pl.BlockSpec((1, _BLK, H // 2), lambda t: (1, t, 0)),
            pl.BlockSpec((_BLK, _K), lambda t: (t, 0)),
        ],
        out_specs=pl.BlockSpec((_BLK, H), lambda t: (t, 0)),
        out_shape=jax.ShapeDtypeStruct((T, H), jnp.float32),
    )(op, op, tw)


@jax.jit
def kernel(hidden_states, gate_up_proj, down_proj, gate_weight):
    B, S, H = hidden_states.shape
    E, _, I2 = gate_up_proj.shape
    T = B * S
    P = (T * _K // _BLK + E) * _BLK

    x = hidden_states.reshape(T, H)
    gu_b = gate_up_proj.astype(jnp.bfloat16)
    dn_b = down_proj.astype(jnp.bfloat16)

    top_i, top_w, xbi, hexcl, tot = _run_router(x, gate_weight, T, H)

    xs, pos0, pos1, bexp = _make_dispatch(T, H, P)(
        top_i.reshape(-1), hexcl.reshape(-1), tot.reshape(-1), xbi)

    os_ = _run_gmm(xs, gu_b, dn_b, bexp, P, H, I2)

    op = _make_gather(T, H, P)(os_, pos0, pos1)
    out = _run_combine(op, top_w, T, H)
    return out.reshape(B, S, H)
